# Initial kernel scaffold; baseline (speedup 1.0000x reference)
import jax
import jax.numpy as jnp
from jax.experimental import pallas as pl


def kernel(x, edge_index, hx, cx, W_g0, b_g0, W_g1, b_g1, W_x2h, b_x2h, W_h2h, b_h2h):
    raise NotImplementedError("write your pallas kernel here")


# speedup vs baseline: 3.3247x; 3.3247x over previous
import jax
import jax.numpy as jnp
from jax import lax
from jax.experimental import pallas as pl
from jax.experimental.pallas import tpu as pltpu
from jax.experimental.pallas import tpu_sc as plsc

N = 10000
E = 320000
D = 128
DH = D // 2
NC = 2
NS = 16
NW = NC * NS
EPW = E // NW
K = 80
NCHUNK = EPW // K
NP = 10240
RPT = NP // NS
ZR = 128
DEGW = 8

_mesh = plsc.VectorSubcoreMesh(core_axis_name="c", subcore_axis_name="s")


def _sc_agg_body(xa_hbm, xb_hbm, src_hbm, dst_hbm, zeros_hbm, ones_hbm,
                 z16_hbm, pa_out, pb_out, deg_out,
                 sidx, didx, rows, ones, zbuf, z16, acc, dacc, sem):
    c = lax.axis_index("c")
    s = lax.axis_index("s")
    wid = s * NC + c
    row0 = s * RPT

    pltpu.sync_copy(zeros_hbm, zbuf)
    pltpu.sync_copy(ones_hbm, ones)
    pltpu.sync_copy(z16_hbm, z16)
    for j in range(RPT // ZR):
        r = row0 + j * ZR
        pltpu.sync_copy(zbuf, acc.at[pl.ds(r, ZR)])
        pltpu.sync_copy(z16, dacc.at[pl.ds(r, ZR)])
    plsc.subcore_barrier()

    def body_a(i, carry):
        base = wid * EPW + i * K
        pltpu.sync_copy(src_hbm.at[pl.ds(base, K)], sidx)
        pltpu.sync_copy(dst_hbm.at[pl.ds(base, K)], didx)
        pltpu.async_copy(xa_hbm.at[sidx], rows, sem).wait()
        pltpu.sync_copy(rows, acc.at[didx], add=True)
        pltpu.sync_copy(ones, dacc.at[didx], add=True)
        return carry

    lax.fori_loop(0, NCHUNK, body_a, 0)
    plsc.subcore_barrier()

    for j in range(RPT // ZR):
        r = row0 + j * ZR
        pltpu.sync_copy(acc.at[pl.ds(r, ZR)], zbuf)
        pltpu.sync_copy(zbuf, pa_out.at[c, pl.ds(r, ZR)])
        pltpu.sync_copy(dacc.at[pl.ds(r, ZR)], z16)
        pltpu.sync_copy(z16, deg_out.at[c, pl.ds(r, ZR)])
    pltpu.sync_copy(zeros_hbm, zbuf)
    for j in range(RPT // ZR):
        r = row0 + j * ZR
        pltpu.sync_copy(zbuf, acc.at[pl.ds(r, ZR)])
    plsc.subcore_barrier()

    def body_b(i, carry):
        base = wid * EPW + i * K
        pltpu.sync_copy(src_hbm.at[pl.ds(base, K)], sidx)
        pltpu.sync_copy(dst_hbm.at[pl.ds(base, K)], didx)
        pltpu.async_copy(xb_hbm.at[sidx], rows, sem).wait()
        pltpu.sync_copy(rows, acc.at[didx], add=True)
        return carry

    lax.fori_loop(0, NCHUNK, body_b, 0)
    plsc.subcore_barrier()

    for j in range(RPT // ZR):
        r = row0 + j * ZR
        pltpu.sync_copy(acc.at[pl.ds(r, ZR)], zbuf)
        pltpu.sync_copy(zbuf, pb_out.at[c, pl.ds(r, ZR)])


def _sc_agg(xa, xb, src, dst, zeros_big, ones8, zeros8):
    out_type = (
        jax.ShapeDtypeStruct((NC, NP, DH), jnp.float32),
        jax.ShapeDtypeStruct((NC, NP, DH), jnp.float32),
        jax.ShapeDtypeStruct((NC, NP, DEGW), jnp.float32),
    )
    scratch = [
        pltpu.VMEM((K,), jnp.int32),
        pltpu.VMEM((K,), jnp.int32),
        pltpu.VMEM((K, DH), jnp.float32),
        pltpu.VMEM((K, DEGW), jnp.float32),
        pltpu.VMEM((ZR, DH), jnp.float32),
        pltpu.VMEM((ZR, DEGW), jnp.float32),
        pltpu.VMEM_SHARED((NP, DH), jnp.float32),
        pltpu.VMEM_SHARED((NP, DEGW), jnp.float32),
        pltpu.SemaphoreType.DMA,
    ]
    params = pltpu.CompilerParams(use_tc_tiling_on_sc=False)
    return pl.kernel(_sc_agg_body, mesh=_mesh, out_type=out_type,
                     scratch_types=scratch, compiler_params=params)(
        xa, xb, src, dst, zeros_big, ones8, zeros8)


BN = 640


def _agg_halves(pa_ref, pb_ref, degp_ref):
    pa = pa_ref[0] + pa_ref[1]
    pb = pb_ref[0] + pb_ref[1]
    dg = degp_ref[0] + degp_ref[1]
    recip = 1.0 / jnp.maximum(dg[:, 0:1], 1.0)
    return pa * recip, pb * recip


def _tc0_body(pa_ref, pb_ref, degp_ref, w_ref, b_ref, outa_ref, outb_ref):
    agg_a, agg_b = _agg_halves(pa_ref, pb_ref, degp_ref)
    h = (jnp.dot(agg_a, w_ref[0:DH, :], preferred_element_type=jnp.float32)
         + jnp.dot(agg_b, w_ref[DH:D, :], preferred_element_type=jnp.float32))
    h = jnp.maximum(h + b_ref[...], 0.0)
    outa_ref[...] = h[:, 0:DH]
    outb_ref[...] = h[:, DH:D]


def _tc_layer0(pa, pb, degp, w, b2):
    grid = (NP // BN,)
    out_shape = (
        jax.ShapeDtypeStruct((NP, DH), jnp.float32),
        jax.ShapeDtypeStruct((NP, DH), jnp.float32),
    )
    return pl.pallas_call(
        _tc0_body,
        grid=grid,
        in_specs=[
            pl.BlockSpec((NC, BN, DH), lambda i: (0, i, 0)),
            pl.BlockSpec((NC, BN, DH), lambda i: (0, i, 0)),
            pl.BlockSpec((NC, BN, DEGW), lambda i: (0, i, 0)),
            pl.BlockSpec((D, D), lambda i: (0, 0)),
            pl.BlockSpec((1, D), lambda i: (0, 0)),
        ],
        out_specs=(
            pl.BlockSpec((BN, DH), lambda i: (i, 0)),
            pl.BlockSpec((BN, DH), lambda i: (i, 0)),
        ),
        out_shape=out_shape,
    )(pa, pb, degp, w, b2)


def _tc1_body(pa_ref, pb_ref, degp_ref, hx_ref, cx_ref, wg_ref, bg_ref,
              wx_ref, wh_ref, bsum_ref, hy_ref, cy_ref):
    agg_a, agg_b = _agg_halves(pa_ref, pb_ref, degp_ref)
    xg = (jnp.dot(agg_a, wg_ref[0:DH, :], preferred_element_type=jnp.float32)
          + jnp.dot(agg_b, wg_ref[DH:D, :], preferred_element_type=jnp.float32)
          + bg_ref[...])
    gates = (jnp.dot(xg, wx_ref[...], preferred_element_type=jnp.float32)
             + jnp.dot(hx_ref[...], wh_ref[...], preferred_element_type=jnp.float32)
             + bsum_ref[...])
    ingate = jax.nn.sigmoid(gates[:, 0:D])
    forgetgate = jax.nn.sigmoid(gates[:, D:2 * D])
    cellgate = jnp.tanh(gates[:, 2 * D:3 * D])
    outgate = jax.nn.sigmoid(gates[:, 3 * D:4 * D])
    cy = cx_ref[...] * forgetgate + ingate * cellgate
    hy_ref[...] = outgate * jnp.tanh(cy)
    cy_ref[...] = cy


def _tc_final(pa, pb, degp, hx, cx, wg, bg2, wx, wh, bsum2):
    grid = (NP // BN,)
    out_shape = (
        jax.ShapeDtypeStruct((NP, D), jnp.float32),
        jax.ShapeDtypeStruct((NP, D), jnp.float32),
    )
    return pl.pallas_call(
        _tc1_body,
        grid=grid,
        in_specs=[
            pl.BlockSpec((NC, BN, DH), lambda i: (0, i, 0)),
            pl.BlockSpec((NC, BN, DH), lambda i: (0, i, 0)),
            pl.BlockSpec((NC, BN, DEGW), lambda i: (0, i, 0)),
            pl.BlockSpec((BN, D), lambda i: (i, 0)),
            pl.BlockSpec((BN, D), lambda i: (i, 0)),
            pl.BlockSpec((D, D), lambda i: (0, 0)),
            pl.BlockSpec((1, D), lambda i: (0, 0)),
            pl.BlockSpec((D, 4 * D), lambda i: (0, 0)),
            pl.BlockSpec((D, 4 * D), lambda i: (0, 0)),
            pl.BlockSpec((1, 4 * D), lambda i: (0, 0)),
        ],
        out_specs=(
            pl.BlockSpec((BN, D), lambda i: (i, 0)),
            pl.BlockSpec((BN, D), lambda i: (i, 0)),
        ),
        out_shape=out_shape,
    )(pa, pb, degp, hx, cx, wg, bg2, wx, wh, bsum2)


def kernel(x, edge_index, hx, cx, W_g0, b_g0, W_g1, b_g1,
           W_x2h, b_x2h, W_h2h, b_h2h):
    src = edge_index[0]
    dst = edge_index[1]
    xa = x[:, 0:DH]
    xb = x[:, DH:D]
    zeros_big = jnp.zeros((ZR, DH), jnp.float32)
    ones8 = jnp.ones((K, DEGW), jnp.float32)
    zeros8 = jnp.zeros((ZR, DEGW), jnp.float32)

    hxp = jnp.pad(hx, ((0, NP - N), (0, 0)))
    cxp = jnp.pad(cx, ((0, NP - N), (0, 0)))

    p0a, p0b, degp = _sc_agg(xa, xb, src, dst, zeros_big, ones8, zeros8)
    h1a, h1b = _tc_layer0(p0a, p0b, degp, W_g0, b_g0.reshape(1, D))
    p1a, p1b, _ = _sc_agg(h1a, h1b, src, dst, zeros_big, ones8, zeros8)
    hy, cy = _tc_final(p1a, p1b, degp, hxp, cxp, W_g1, b_g1.reshape(1, D),
                       W_x2h, W_h2h, (b_x2h + b_h2h).reshape(1, 4 * D))
    return (hy[:N], cy[:N])


# speedup vs baseline: 9.4027x; 2.8282x over previous
import jax
import jax.numpy as jnp
from jax import lax
from jax.experimental import pallas as pl
from jax.experimental.pallas import tpu as pltpu
from jax.experimental.pallas import tpu_sc as plsc

N = 10000
E = 320000
D = 128
DH = D // 2
NC = 2
NS = 16
NW = NC * NS
EPW = E // NW
K = 100
NCHUNK = EPW // K
NBUF = 5
NP = 10240
RPT = NP // NS
ZR = 128
DEGW = 8

_mesh = plsc.VectorSubcoreMesh(core_axis_name="c", subcore_axis_name="s")


def _sc_agg_body(xa_hbm, xb_hbm, src2_hbm, dst2_hbm, zeros_hbm, ones_hbm,
                 z16_hbm, pa_out, pb_out, deg_out,
                 sidx2, didx2, rows, ones, zbuf, z16, acc, dacc,
                 gsem, ssem, osem):
    c = lax.axis_index("c")
    s = lax.axis_index("s")
    wid = s * NC + c
    row0 = s * RPT
    crow0 = wid * NCHUNK

    pltpu.sync_copy(zeros_hbm, zbuf)
    pltpu.sync_copy(ones_hbm, ones)
    pltpu.sync_copy(z16_hbm, z16)
    pltpu.sync_copy(src2_hbm.at[pl.ds(crow0, NCHUNK)], sidx2)
    pltpu.sync_copy(dst2_hbm.at[pl.ds(crow0, NCHUNK)], didx2)
    for j in range(RPT // ZR):
        r = row0 + j * ZR
        pltpu.sync_copy(zbuf, acc.at[pl.ds(r, ZR)])
        pltpu.sync_copy(z16, dacc.at[pl.ds(r, ZR)])
    plsc.subcore_barrier()

    def run_pass(table, with_deg):
        for b in range(NBUF):
            pltpu.async_copy(table.at[sidx2.at[b]], rows.at[b], gsem.at[b])

        def outer(it, carry):
            base = it * NBUF
            for b in range(NBUF):
                g = base + b
                pltpu.make_async_copy(
                    table.at[sidx2.at[g]], rows.at[b], gsem.at[b]).wait()
                pltpu.async_copy(rows.at[b], acc.at[didx2.at[g]],
                                 ssem.at[b], add=True)
                if with_deg:
                    pltpu.async_copy(ones, dacc.at[didx2.at[g]],
                                     osem.at[b], add=True)
            for b in range(NBUF):
                g = base + b
                pltpu.make_async_copy(rows.at[b], acc.at[didx2.at[g]],
                                      ssem.at[b]).wait()
                if with_deg:
                    pltpu.make_async_copy(ones, dacc.at[didx2.at[g]],
                                          osem.at[b]).wait()

                @pl.when(g + NBUF < NCHUNK)
                def _():
                    pltpu.async_copy(table.at[sidx2.at[g + NBUF]],
                                     rows.at[b], gsem.at[b])
            return carry

        lax.fori_loop(0, NCHUNK // NBUF, outer, 0)

    run_pass(xa_hbm, True)
    plsc.subcore_barrier()

    for j in range(RPT // ZR):
        r = row0 + j * ZR
        pltpu.sync_copy(acc.at[pl.ds(r, ZR)], zbuf)
        pltpu.sync_copy(zbuf, pa_out.at[c, pl.ds(r, ZR)])
        pltpu.sync_copy(dacc.at[pl.ds(r, ZR)], z16)
        pltpu.sync_copy(z16, deg_out.at[c, pl.ds(r, ZR)])
    pltpu.sync_copy(zeros_hbm, zbuf)
    for j in range(RPT // ZR):
        r = row0 + j * ZR
        pltpu.sync_copy(zbuf, acc.at[pl.ds(r, ZR)])
    plsc.subcore_barrier()

    run_pass(xb_hbm, False)
    plsc.subcore_barrier()

    for j in range(RPT // ZR):
        r = row0 + j * ZR
        pltpu.sync_copy(acc.at[pl.ds(r, ZR)], zbuf)
        pltpu.sync_copy(zbuf, pb_out.at[c, pl.ds(r, ZR)])


def _sc_agg(xa, xb, src, dst, zeros_big, ones8, zeros8):
    out_type = (
        jax.ShapeDtypeStruct((NC, NP, DH), jnp.float32),
        jax.ShapeDtypeStruct((NC, NP, DH), jnp.float32),
        jax.ShapeDtypeStruct((NC, NP, DEGW), jnp.float32),
    )
    scratch = [
        pltpu.VMEM((NCHUNK, K), jnp.int32),
        pltpu.VMEM((NCHUNK, K), jnp.int32),
        pltpu.VMEM((NBUF, K, DH), jnp.float32),
        pltpu.VMEM((K, DEGW), jnp.float32),
        pltpu.VMEM((ZR, DH), jnp.float32),
        pltpu.VMEM((ZR, DEGW), jnp.float32),
        pltpu.VMEM_SHARED((NP, DH), jnp.float32),
        pltpu.VMEM_SHARED((NP, DEGW), jnp.float32),
        pltpu.SemaphoreType.DMA((NBUF,)),
        pltpu.SemaphoreType.DMA((NBUF,)),
        pltpu.SemaphoreType.DMA((NBUF,)),
    ]
    params = pltpu.CompilerParams(use_tc_tiling_on_sc=False)
    return pl.kernel(_sc_agg_body, mesh=_mesh, out_type=out_type,
                     scratch_types=scratch, compiler_params=params)(
        xa, xb, src, dst, zeros_big, ones8, zeros8)


BN = 640


def _agg_halves(pa_ref, pb_ref, degp_ref):
    pa = pa_ref[0] + pa_ref[1]
    pb = pb_ref[0] + pb_ref[1]
    dg = degp_ref[0] + degp_ref[1]
    recip = 1.0 / jnp.maximum(dg[:, 0:1], 1.0)
    return pa * recip, pb * recip


def _tc0_body(pa_ref, pb_ref, degp_ref, w_ref, b_ref, outa_ref, outb_ref):
    agg_a, agg_b = _agg_halves(pa_ref, pb_ref, degp_ref)
    h = (jnp.dot(agg_a, w_ref[0:DH, :], preferred_element_type=jnp.float32)
         + jnp.dot(agg_b, w_ref[DH:D, :], preferred_element_type=jnp.float32))
    h = jnp.maximum(h + b_ref[...], 0.0)
    outa_ref[...] = h[:, 0:DH]
    outb_ref[...] = h[:, DH:D]


def _tc_layer0(pa, pb, degp, w, b2):
    grid = (NP // BN,)
    out_shape = (
        jax.ShapeDtypeStruct((NP, DH), jnp.float32),
        jax.ShapeDtypeStruct((NP, DH), jnp.float32),
    )
    return pl.pallas_call(
        _tc0_body,
        grid=grid,
        in_specs=[
            pl.BlockSpec((NC, BN, DH), lambda i: (0, i, 0)),
            pl.BlockSpec((NC, BN, DH), lambda i: (0, i, 0)),
            pl.BlockSpec((NC, BN, DEGW), lambda i: (0, i, 0)),
            pl.BlockSpec((D, D), lambda i: (0, 0)),
            pl.BlockSpec((1, D), lambda i: (0, 0)),
        ],
        out_specs=(
            pl.BlockSpec((BN, DH), lambda i: (i, 0)),
            pl.BlockSpec((BN, DH), lambda i: (i, 0)),
        ),
        out_shape=out_shape,
    )(pa, pb, degp, w, b2)


def _tc1_body(pa_ref, pb_ref, degp_ref, hx_ref, cx_ref, wg_ref, bg_ref,
              wx_ref, wh_ref, bsum_ref, hy_ref, cy_ref):
    agg_a, agg_b = _agg_halves(pa_ref, pb_ref, degp_ref)
    xg = (jnp.dot(agg_a, wg_ref[0:DH, :], preferred_element_type=jnp.float32)
          + jnp.dot(agg_b, wg_ref[DH:D, :], preferred_element_type=jnp.float32)
          + bg_ref[...])
    gates = (jnp.dot(xg, wx_ref[...], preferred_element_type=jnp.float32)
             + jnp.dot(hx_ref[...], wh_ref[...], preferred_element_type=jnp.float32)
             + bsum_ref[...])
    ingate = jax.nn.sigmoid(gates[:, 0:D])
    forgetgate = jax.nn.sigmoid(gates[:, D:2 * D])
    cellgate = jnp.tanh(gates[:, 2 * D:3 * D])
    outgate = jax.nn.sigmoid(gates[:, 3 * D:4 * D])
    cy = cx_ref[...] * forgetgate + ingate * cellgate
    hy_ref[...] = outgate * jnp.tanh(cy)
    cy_ref[...] = cy


def _tc_final(pa, pb, degp, hx, cx, wg, bg2, wx, wh, bsum2):
    grid = (NP // BN,)
    out_shape = (
        jax.ShapeDtypeStruct((NP, D), jnp.float32),
        jax.ShapeDtypeStruct((NP, D), jnp.float32),
    )
    return pl.pallas_call(
        _tc1_body,
        grid=grid,
        in_specs=[
            pl.BlockSpec((NC, BN, DH), lambda i: (0, i, 0)),
            pl.BlockSpec((NC, BN, DH), lambda i: (0, i, 0)),
            pl.BlockSpec((NC, BN, DEGW), lambda i: (0, i, 0)),
            pl.BlockSpec((BN, D), lambda i: (i, 0)),
            pl.BlockSpec((BN, D), lambda i: (i, 0)),
            pl.BlockSpec((D, D), lambda i: (0, 0)),
            pl.BlockSpec((1, D), lambda i: (0, 0)),
            pl.BlockSpec((D, 4 * D), lambda i: (0, 0)),
            pl.BlockSpec((D, 4 * D), lambda i: (0, 0)),
            pl.BlockSpec((1, 4 * D), lambda i: (0, 0)),
        ],
        out_specs=(
            pl.BlockSpec((BN, D), lambda i: (i, 0)),
            pl.BlockSpec((BN, D), lambda i: (i, 0)),
        ),
        out_shape=out_shape,
    )(pa, pb, degp, hx, cx, wg, bg2, wx, wh, bsum2)


def kernel(x, edge_index, hx, cx, W_g0, b_g0, W_g1, b_g1,
           W_x2h, b_x2h, W_h2h, b_h2h):
    src = edge_index[0].reshape(E // K, K)
    dst = edge_index[1].reshape(E // K, K)
    xa = x[:, 0:DH]
    xb = x[:, DH:D]
    zeros_big = jnp.zeros((ZR, DH), jnp.float32)
    ones8 = jnp.ones((K, DEGW), jnp.float32)
    zeros8 = jnp.zeros((ZR, DEGW), jnp.float32)

    hxp = jnp.pad(hx, ((0, NP - N), (0, 0)))
    cxp = jnp.pad(cx, ((0, NP - N), (0, 0)))

    p0a, p0b, degp = _sc_agg(xa, xb, src, dst, zeros_big, ones8, zeros8)
    h1a, h1b = _tc_layer0(p0a, p0b, degp, W_g0, b_g0.reshape(1, D))
    p1a, p1b, _ = _sc_agg(h1a, h1b, src, dst, zeros_big, ones8, zeros8)
    hy, cy = _tc_final(p1a, p1b, degp, hxp, cxp, W_g1, b_g1.reshape(1, D),
                       W_x2h, W_h2h, (b_x2h + b_h2h).reshape(1, 4 * D))
    return (hy[:N], cy[:N])


# speedup vs baseline: 9.6040x; 1.0214x over previous
import functools

import jax
import jax.numpy as jnp
from jax import lax
from jax.experimental import pallas as pl
from jax.experimental.pallas import tpu as pltpu
from jax.experimental.pallas import tpu_sc as plsc

N = 10000
E = 320000
D = 128
DH = D // 2
NC = 2
NS = 16
NW = NC * NS
EPW = E // NW
K = 100
NCHUNK = EPW // K
NBUF = 5
NP = 10240
RPT = NP // NS
ZR = 128
DEGW = 8

_mesh = plsc.VectorSubcoreMesh(core_axis_name="c", subcore_axis_name="s")


def _sc_agg_body(with_deg, *refs):
    if with_deg:
        (xa_hbm, xb_hbm, src2_hbm, dst2_hbm, zeros_hbm, ones_hbm,
         z16_hbm, pa_out, pb_out, deg_out,
         sidx2, didx2, rows, ones, zbuf, z16, acc, dacc,
         gsem, ssem, osem) = refs
    else:
        (xa_hbm, xb_hbm, src2_hbm, dst2_hbm, zeros_hbm,
         pa_out, pb_out,
         sidx2, didx2, rows, zbuf, acc,
         gsem, ssem) = refs
    c = lax.axis_index("c")
    s = lax.axis_index("s")
    wid = s * NC + c
    row0 = s * RPT
    crow0 = wid * NCHUNK

    pltpu.sync_copy(zeros_hbm, zbuf)
    if with_deg:
        pltpu.sync_copy(ones_hbm, ones)
        pltpu.sync_copy(z16_hbm, z16)
    pltpu.sync_copy(src2_hbm.at[pl.ds(crow0, NCHUNK)], sidx2)
    pltpu.sync_copy(dst2_hbm.at[pl.ds(crow0, NCHUNK)], didx2)
    for j in range(RPT // ZR):
        r = row0 + j * ZR
        pltpu.sync_copy(zbuf, acc.at[pl.ds(r, ZR)])
        if with_deg:
            pltpu.sync_copy(z16, dacc.at[pl.ds(r, ZR)])
    plsc.subcore_barrier()

    def run_pass(table, deg_pass):
        for b in range(NBUF):
            pltpu.async_copy(table.at[sidx2.at[b]], rows.at[b], gsem.at[b])

        def outer(it, carry):
            base = it * NBUF
            for b in range(NBUF):
                g = base + b
                pltpu.make_async_copy(
                    table.at[sidx2.at[g]], rows.at[b], gsem.at[b]).wait()
                pltpu.async_copy(rows.at[b], acc.at[didx2.at[g]],
                                 ssem.at[b], add=True)
                if deg_pass:
                    pltpu.async_copy(ones, dacc.at[didx2.at[g]],
                                     osem.at[b], add=True)
            for b in range(NBUF):
                g = base + b
                pltpu.make_async_copy(rows.at[b], acc.at[didx2.at[g]],
                                      ssem.at[b]).wait()
                if deg_pass:
                    pltpu.make_async_copy(ones, dacc.at[didx2.at[g]],
                                          osem.at[b]).wait()

                @pl.when(g + NBUF < NCHUNK)
                def _():
                    pltpu.async_copy(table.at[sidx2.at[g + NBUF]],
                                     rows.at[b], gsem.at[b])
            return carry

        lax.fori_loop(0, NCHUNK // NBUF, outer, 0)

    run_pass(xa_hbm, with_deg)
    plsc.subcore_barrier()

    for j in range(RPT // ZR):
        r = row0 + j * ZR
        pltpu.sync_copy(acc.at[pl.ds(r, ZR)], zbuf)
        pltpu.sync_copy(zbuf, pa_out.at[c, pl.ds(r, ZR)])
        if with_deg:
            pltpu.sync_copy(dacc.at[pl.ds(r, ZR)], z16)
            pltpu.sync_copy(z16, deg_out.at[c, pl.ds(r, ZR)])
    pltpu.sync_copy(zeros_hbm, zbuf)
    for j in range(RPT // ZR):
        r = row0 + j * ZR
        pltpu.sync_copy(zbuf, acc.at[pl.ds(r, ZR)])
    plsc.subcore_barrier()

    run_pass(xb_hbm, False)
    plsc.subcore_barrier()

    for j in range(RPT // ZR):
        r = row0 + j * ZR
        pltpu.sync_copy(acc.at[pl.ds(r, ZR)], zbuf)
        pltpu.sync_copy(zbuf, pb_out.at[c, pl.ds(r, ZR)])


def _sc_agg_deg(xa, xb, src2, dst2, zeros_big, ones8, zeros8):
    out_type = (
        jax.ShapeDtypeStruct((NC, NP, DH), jnp.float32),
        jax.ShapeDtypeStruct((NC, NP, DH), jnp.float32),
        jax.ShapeDtypeStruct((NC, NP, DEGW), jnp.float32),
    )
    scratch = [
        pltpu.VMEM((NCHUNK, K), jnp.int32),
        pltpu.VMEM((NCHUNK, K), jnp.int32),
        pltpu.VMEM((NBUF, K, DH), jnp.float32),
        pltpu.VMEM((K, DEGW), jnp.float32),
        pltpu.VMEM((ZR, DH), jnp.float32),
        pltpu.VMEM((ZR, DEGW), jnp.float32),
        pltpu.VMEM_SHARED((NP, DH), jnp.float32),
        pltpu.VMEM_SHARED((NP, DEGW), jnp.float32),
        pltpu.SemaphoreType.DMA((NBUF,)),
        pltpu.SemaphoreType.DMA((NBUF,)),
        pltpu.SemaphoreType.DMA((NBUF,)),
    ]
    params = pltpu.CompilerParams(use_tc_tiling_on_sc=False)
    fn = functools.partial(_sc_agg_body, True)
    return pl.kernel(fn, mesh=_mesh, out_type=out_type,
                     scratch_types=scratch, compiler_params=params)(
        xa, xb, src2, dst2, zeros_big, ones8, zeros8)


def _sc_agg(xa, xb, src2, dst2, zeros_big):
    out_type = (
        jax.ShapeDtypeStruct((NC, NP, DH), jnp.float32),
        jax.ShapeDtypeStruct((NC, NP, DH), jnp.float32),
    )
    scratch = [
        pltpu.VMEM((NCHUNK, K), jnp.int32),
        pltpu.VMEM((NCHUNK, K), jnp.int32),
        pltpu.VMEM((NBUF, K, DH), jnp.float32),
        pltpu.VMEM((ZR, DH), jnp.float32),
        pltpu.VMEM_SHARED((NP, DH), jnp.float32),
        pltpu.SemaphoreType.DMA((NBUF,)),
        pltpu.SemaphoreType.DMA((NBUF,)),
    ]
    params = pltpu.CompilerParams(use_tc_tiling_on_sc=False)
    fn = functools.partial(_sc_agg_body, False)
    return pl.kernel(fn, mesh=_mesh, out_type=out_type,
                     scratch_types=scratch, compiler_params=params)(
        xa, xb, src2, dst2, zeros_big)


BN = 400


def _agg_halves(pa_ref, pb_ref, degp_ref):
    pa = pa_ref[0] + pa_ref[1]
    pb = pb_ref[0] + pb_ref[1]
    dg = degp_ref[0] + degp_ref[1]
    recip = 1.0 / jnp.maximum(dg[:, 0:1], 1.0)
    return pa * recip, pb * recip


def _tc0_body(pa_ref, pb_ref, degp_ref, w_ref, b_ref, outa_ref, outb_ref):
    agg_a, agg_b = _agg_halves(pa_ref, pb_ref, degp_ref)
    h = (jnp.dot(agg_a, w_ref[0:DH, :], preferred_element_type=jnp.float32)
         + jnp.dot(agg_b, w_ref[DH:D, :], preferred_element_type=jnp.float32))
    h = jnp.maximum(h + b_ref[...], 0.0)
    outa_ref[...] = h[:, 0:DH]
    outb_ref[...] = h[:, DH:D]


def _tc_layer0(pa, pb, degp, w, b2):
    grid = (N // BN,)
    out_shape = (
        jax.ShapeDtypeStruct((N, DH), jnp.float32),
        jax.ShapeDtypeStruct((N, DH), jnp.float32),
    )
    return pl.pallas_call(
        _tc0_body,
        grid=grid,
        in_specs=[
            pl.BlockSpec((NC, BN, DH), lambda i: (0, i, 0)),
            pl.BlockSpec((NC, BN, DH), lambda i: (0, i, 0)),
            pl.BlockSpec((NC, BN, DEGW), lambda i: (0, i, 0)),
            pl.BlockSpec((D, D), lambda i: (0, 0)),
            pl.BlockSpec((1, D), lambda i: (0, 0)),
        ],
        out_specs=(
            pl.BlockSpec((BN, DH), lambda i: (i, 0)),
            pl.BlockSpec((BN, DH), lambda i: (i, 0)),
        ),
        out_shape=out_shape,
    )(pa, pb, degp, w, b2)


def _tc1_body(pa_ref, pb_ref, degp_ref, hx_ref, cx_ref, wg_ref, bg_ref,
              wx_ref, wh_ref, bsum_ref, hy_ref, cy_ref):
    agg_a, agg_b = _agg_halves(pa_ref, pb_ref, degp_ref)
    xg = (jnp.dot(agg_a, wg_ref[0:DH, :], preferred_element_type=jnp.float32)
          + jnp.dot(agg_b, wg_ref[DH:D, :], preferred_element_type=jnp.float32)
          + bg_ref[...])
    gates = (jnp.dot(xg, wx_ref[...], preferred_element_type=jnp.float32)
             + jnp.dot(hx_ref[...], wh_ref[...], preferred_element_type=jnp.float32)
             + bsum_ref[...])
    ingate = jax.nn.sigmoid(gates[:, 0:D])
    forgetgate = jax.nn.sigmoid(gates[:, D:2 * D])
    cellgate = jnp.tanh(gates[:, 2 * D:3 * D])
    outgate = jax.nn.sigmoid(gates[:, 3 * D:4 * D])
    cy = cx_ref[...] * forgetgate + ingate * cellgate
    hy_ref[...] = outgate * jnp.tanh(cy)
    cy_ref[...] = cy


def _tc_final(pa, pb, degp, hx, cx, wg, bg2, wx, wh, bsum2):
    grid = (N // BN,)
    out_shape = (
        jax.ShapeDtypeStruct((N, D), jnp.float32),
        jax.ShapeDtypeStruct((N, D), jnp.float32),
    )
    return pl.pallas_call(
        _tc1_body,
        grid=grid,
        in_specs=[
            pl.BlockSpec((NC, BN, DH), lambda i: (0, i, 0)),
            pl.BlockSpec((NC, BN, DH), lambda i: (0, i, 0)),
            pl.BlockSpec((NC, BN, DEGW), lambda i: (0, i, 0)),
            pl.BlockSpec((BN, D), lambda i: (i, 0)),
            pl.BlockSpec((BN, D), lambda i: (i, 0)),
            pl.BlockSpec((D, D), lambda i: (0, 0)),
            pl.BlockSpec((1, D), lambda i: (0, 0)),
            pl.BlockSpec((D, 4 * D), lambda i: (0, 0)),
            pl.BlockSpec((D, 4 * D), lambda i: (0, 0)),
            pl.BlockSpec((1, 4 * D), lambda i: (0, 0)),
        ],
        out_specs=(
            pl.BlockSpec((BN, D), lambda i: (i, 0)),
            pl.BlockSpec((BN, D), lambda i: (i, 0)),
        ),
        out_shape=out_shape,
    )(pa, pb, degp, hx, cx, wg, bg2, wx, wh, bsum2)


def kernel(x, edge_index, hx, cx, W_g0, b_g0, W_g1, b_g1,
           W_x2h, b_x2h, W_h2h, b_h2h):
    src2 = edge_index[0].reshape(E // K, K)
    dst2 = edge_index[1].reshape(E // K, K)
    xa = x[:, 0:DH]
    xb = x[:, DH:D]
    zeros_big = jnp.zeros((ZR, DH), jnp.float32)
    ones8 = jnp.ones((K, DEGW), jnp.float32)
    zeros8 = jnp.zeros((ZR, DEGW), jnp.float32)

    p0a, p0b, degp = _sc_agg_deg(xa, xb, src2, dst2, zeros_big, ones8, zeros8)
    h1a, h1b = _tc_layer0(p0a, p0b, degp, W_g0, b_g0.reshape(1, D))
    p1a, p1b = _sc_agg(h1a, h1b, src2, dst2, zeros_big)
    hy, cy = _tc_final(p1a, p1b, degp, hx, cx, W_g1, b_g1.reshape(1, D),
                       W_x2h, W_h2h, (b_x2h + b_h2h).reshape(1, 4 * D))
    return (hy, cy)


# speedup vs baseline: 10.0106x; 1.0423x over previous
import functools

import jax
import jax.numpy as jnp
from jax import lax
from jax.experimental import pallas as pl
from jax.experimental.pallas import tpu as pltpu
from jax.experimental.pallas import tpu_sc as plsc

N = 10000
E = 320000
D = 128
DH = D // 2
NC = 2
NS = 16
NW = NC * NS
EPW = E // NW
K = 100
NCHUNK = EPW // K
NBUF = 5
NP = 10240
RPT = NP // NS
ZR = 128
DEGW = 8

_mesh = plsc.VectorSubcoreMesh(core_axis_name="c", subcore_axis_name="s")


def _sc_agg_body(with_deg, *refs):
    if with_deg:
        (xa_hbm, xb_hbm, src2_hbm, dst2_hbm, zeros_hbm, ones_hbm,
         z16_hbm, pa_out, pb_out, deg_out,
         sidx2, didx2, rows, ones, zbuf, z16, acc, dacc,
         gsem, ssem, osem) = refs
    else:
        (xa_hbm, xb_hbm, src2_hbm, dst2_hbm, zeros_hbm,
         pa_out, pb_out,
         sidx2, didx2, rows, zbuf, acc,
         gsem, ssem) = refs
    c = lax.axis_index("c")
    s = lax.axis_index("s")
    wid = s * NC + c
    row0 = s * RPT
    crow0 = wid * NCHUNK

    pltpu.sync_copy(zeros_hbm, zbuf)
    if with_deg:
        pltpu.sync_copy(ones_hbm, ones)
        pltpu.sync_copy(z16_hbm, z16)
    pltpu.sync_copy(src2_hbm.at[pl.ds(crow0, NCHUNK)], sidx2)
    pltpu.sync_copy(dst2_hbm.at[pl.ds(crow0, NCHUNK)], didx2)
    for j in range(RPT // ZR):
        r = row0 + j * ZR
        pltpu.sync_copy(zbuf, acc.at[pl.ds(r, ZR)])
        if with_deg:
            pltpu.sync_copy(z16, dacc.at[pl.ds(r, ZR)])
    plsc.subcore_barrier()

    def run_pass(table, deg_pass):
        for b in range(NBUF):
            pltpu.async_copy(table.at[sidx2.at[b]], rows.at[b], gsem.at[b])

        def outer(it, carry):
            base = it * NBUF
            for b in range(NBUF):
                g = base + b
                pltpu.make_async_copy(
                    table.at[sidx2.at[g]], rows.at[b], gsem.at[b]).wait()
                pltpu.async_copy(rows.at[b], acc.at[didx2.at[g]],
                                 ssem.at[b], add=True)
                if deg_pass:
                    pltpu.async_copy(ones, dacc.at[didx2.at[g]],
                                     osem.at[b], add=True)
            for b in range(NBUF):
                g = base + b
                pltpu.make_async_copy(rows.at[b], acc.at[didx2.at[g]],
                                      ssem.at[b]).wait()
                if deg_pass:
                    pltpu.make_async_copy(ones, dacc.at[didx2.at[g]],
                                          osem.at[b]).wait()

                @pl.when(g + NBUF < NCHUNK)
                def _():
                    pltpu.async_copy(table.at[sidx2.at[g + NBUF]],
                                     rows.at[b], gsem.at[b])
            return carry

        lax.fori_loop(0, NCHUNK // NBUF, outer, 0)

    run_pass(xa_hbm, with_deg)
    plsc.subcore_barrier()

    for j in range(RPT // ZR):
        r = row0 + j * ZR
        pltpu.sync_copy(acc.at[pl.ds(r, ZR)], zbuf)
        pltpu.sync_copy(zbuf, pa_out.at[c, pl.ds(r, ZR)])
        if with_deg:
            pltpu.sync_copy(dacc.at[pl.ds(r, ZR)], z16)
            pltpu.sync_copy(z16, deg_out.at[c, pl.ds(r, ZR)])
    pltpu.sync_copy(zeros_hbm, zbuf)
    for j in range(RPT // ZR):
        r = row0 + j * ZR
        pltpu.sync_copy(zbuf, acc.at[pl.ds(r, ZR)])
    plsc.subcore_barrier()

    run_pass(xb_hbm, False)
    plsc.subcore_barrier()

    for j in range(RPT // ZR):
        r = row0 + j * ZR
        pltpu.sync_copy(acc.at[pl.ds(r, ZR)], zbuf)
        pltpu.sync_copy(zbuf, pb_out.at[c, pl.ds(r, ZR)])


def _sc_agg_deg(xa, xb, src2, dst2, zeros_big, ones8, zeros8):
    out_type = (
        jax.ShapeDtypeStruct((NC, NP, DH), jnp.float32),
        jax.ShapeDtypeStruct((NC, NP, DH), jnp.float32),
        jax.ShapeDtypeStruct((NC, NP, DEGW), jnp.float32),
    )
    scratch = [
        pltpu.VMEM((NCHUNK, K), jnp.int32),
        pltpu.VMEM((NCHUNK, K), jnp.int32),
        pltpu.VMEM((NBUF, K, DH), jnp.float32),
        pltpu.VMEM((K, DEGW), jnp.float32),
        pltpu.VMEM((ZR, DH), jnp.float32),
        pltpu.VMEM((ZR, DEGW), jnp.float32),
        pltpu.VMEM_SHARED((NP, DH), jnp.float32),
        pltpu.VMEM_SHARED((NP, DEGW), jnp.float32),
        pltpu.SemaphoreType.DMA((NBUF,)),
        pltpu.SemaphoreType.DMA((NBUF,)),
        pltpu.SemaphoreType.DMA((NBUF,)),
    ]
    params = pltpu.CompilerParams(use_tc_tiling_on_sc=False)
    fn = functools.partial(_sc_agg_body, True)
    return pl.kernel(fn, mesh=_mesh, out_type=out_type,
                     scratch_types=scratch, compiler_params=params)(
        xa, xb, src2, dst2, zeros_big, ones8, zeros8)


def _sc_agg(xa, xb, src2, dst2, zeros_big):
    out_type = (
        jax.ShapeDtypeStruct((NC, NP, DH), jnp.float32),
        jax.ShapeDtypeStruct((NC, NP, DH), jnp.float32),
    )
    scratch = [
        pltpu.VMEM((NCHUNK, K), jnp.int32),
        pltpu.VMEM((NCHUNK, K), jnp.int32),
        pltpu.VMEM((NBUF, K, DH), jnp.float32),
        pltpu.VMEM((ZR, DH), jnp.float32),
        pltpu.VMEM_SHARED((NP, DH), jnp.float32),
        pltpu.SemaphoreType.DMA((NBUF,)),
        pltpu.SemaphoreType.DMA((NBUF,)),
    ]
    params = pltpu.CompilerParams(use_tc_tiling_on_sc=False)
    fn = functools.partial(_sc_agg_body, False)
    return pl.kernel(fn, mesh=_mesh, out_type=out_type,
                     scratch_types=scratch, compiler_params=params)(
        xa, xb, src2, dst2, zeros_big)


BN = 1000


def _agg_halves(pa_ref, pb_ref, degp_ref):
    pa = pa_ref[0] + pa_ref[1]
    pb = pb_ref[0] + pb_ref[1]
    dg = degp_ref[0] + degp_ref[1]
    recip = 1.0 / jnp.maximum(dg[:, 0:1], 1.0)
    return pa * recip, pb * recip


def _tc0_body(pa_ref, pb_ref, degp_ref, w_ref, b_ref, outa_ref, outb_ref):
    agg_a, agg_b = _agg_halves(pa_ref, pb_ref, degp_ref)
    h = (jnp.dot(agg_a, w_ref[0:DH, :], preferred_element_type=jnp.float32)
         + jnp.dot(agg_b, w_ref[DH:D, :], preferred_element_type=jnp.float32))
    h = jnp.maximum(h + b_ref[...], 0.0)
    outa_ref[...] = h[:, 0:DH]
    outb_ref[...] = h[:, DH:D]


def _tc_layer0(pa, pb, degp, w, b2):
    grid = (N // BN,)
    out_shape = (
        jax.ShapeDtypeStruct((N, DH), jnp.float32),
        jax.ShapeDtypeStruct((N, DH), jnp.float32),
    )
    return pl.pallas_call(
        _tc0_body,
        grid=grid,
        in_specs=[
            pl.BlockSpec((NC, BN, DH), lambda i: (0, i, 0)),
            pl.BlockSpec((NC, BN, DH), lambda i: (0, i, 0)),
            pl.BlockSpec((NC, BN, DEGW), lambda i: (0, i, 0)),
            pl.BlockSpec((D, D), lambda i: (0, 0)),
            pl.BlockSpec((1, D), lambda i: (0, 0)),
        ],
        out_specs=(
            pl.BlockSpec((BN, DH), lambda i: (i, 0)),
            pl.BlockSpec((BN, DH), lambda i: (i, 0)),
        ),
        out_shape=out_shape,
    )(pa, pb, degp, w, b2)


def _tc1_body(pa_ref, pb_ref, degp_ref, hx_ref, cx_ref, wg_ref, bg_ref,
              wx_ref, wh_ref, bsum_ref, hy_ref, cy_ref):
    agg_a, agg_b = _agg_halves(pa_ref, pb_ref, degp_ref)
    xg = (jnp.dot(agg_a, wg_ref[0:DH, :], preferred_element_type=jnp.float32)
          + jnp.dot(agg_b, wg_ref[DH:D, :], preferred_element_type=jnp.float32)
          + bg_ref[...])
    gates = (jnp.dot(xg, wx_ref[...], preferred_element_type=jnp.float32)
             + jnp.dot(hx_ref[...], wh_ref[...], preferred_element_type=jnp.float32)
             + bsum_ref[...])
    ingate = jax.nn.sigmoid(gates[:, 0:D])
    forgetgate = jax.nn.sigmoid(gates[:, D:2 * D])
    cellgate = jnp.tanh(gates[:, 2 * D:3 * D])
    outgate = jax.nn.sigmoid(gates[:, 3 * D:4 * D])
    cy = cx_ref[...] * forgetgate + ingate * cellgate
    hy_ref[...] = outgate * jnp.tanh(cy)
    cy_ref[...] = cy


def _tc_final(pa, pb, degp, hx, cx, wg, bg2, wx, wh, bsum2):
    grid = (N // BN,)
    out_shape = (
        jax.ShapeDtypeStruct((N, D), jnp.float32),
        jax.ShapeDtypeStruct((N, D), jnp.float32),
    )
    return pl.pallas_call(
        _tc1_body,
        grid=grid,
        in_specs=[
            pl.BlockSpec((NC, BN, DH), lambda i: (0, i, 0)),
            pl.BlockSpec((NC, BN, DH), lambda i: (0, i, 0)),
            pl.BlockSpec((NC, BN, DEGW), lambda i: (0, i, 0)),
            pl.BlockSpec((BN, D), lambda i: (i, 0)),
            pl.BlockSpec((BN, D), lambda i: (i, 0)),
            pl.BlockSpec((D, D), lambda i: (0, 0)),
            pl.BlockSpec((1, D), lambda i: (0, 0)),
            pl.BlockSpec((D, 4 * D), lambda i: (0, 0)),
            pl.BlockSpec((D, 4 * D), lambda i: (0, 0)),
            pl.BlockSpec((1, 4 * D), lambda i: (0, 0)),
        ],
        out_specs=(
            pl.BlockSpec((BN, D), lambda i: (i, 0)),
            pl.BlockSpec((BN, D), lambda i: (i, 0)),
        ),
        out_shape=out_shape,
    )(pa, pb, degp, hx, cx, wg, bg2, wx, wh, bsum2)


def kernel(x, edge_index, hx, cx, W_g0, b_g0, W_g1, b_g1,
           W_x2h, b_x2h, W_h2h, b_h2h):
    src2 = edge_index[0].reshape(E // K, K)
    dst2 = edge_index[1].reshape(E // K, K)
    xa = x[:, 0:DH]
    xb = x[:, DH:D]
    zeros_big = jnp.zeros((ZR, DH), jnp.float32)
    ones8 = jnp.ones((K, DEGW), jnp.float32)
    zeros8 = jnp.zeros((ZR, DEGW), jnp.float32)

    p0a, p0b, degp = _sc_agg_deg(xa, xb, src2, dst2, zeros_big, ones8, zeros8)
    h1a, h1b = _tc_layer0(p0a, p0b, degp, W_g0, b_g0.reshape(1, D))
    p1a, p1b = _sc_agg(h1a, h1b, src2, dst2, zeros_big)
    hy, cy = _tc_final(p1a, p1b, degp, hx, cx, W_g1, b_g1.reshape(1, D),
                       W_x2h, W_h2h, (b_x2h + b_h2h).reshape(1, 4 * D))
    return (hy, cy)


# speedup vs baseline: 11.7808x; 1.1768x over previous
import functools

import jax
import jax.numpy as jnp
from jax import lax
from jax.experimental import pallas as pl
from jax.experimental.pallas import tpu as pltpu
from jax.experimental.pallas import tpu_sc as plsc

N = 10000
E = 320000
D = 128
DH = D // 2
NC = 2
NS = 16
NW = NC * NS
EPW = E // NW
K = 100
NCHUNK = EPW // K
NBUF = 5
NP = 10240
RPT = NP // NS
ZR = 128
DEGW = 8

_mesh = plsc.VectorSubcoreMesh(core_axis_name="c", subcore_axis_name="s")


def _sc_agg_body(with_deg, *refs):
    if with_deg:
        (xr_hbm, sa2_hbm, sb2_hbm, dst2_hbm, zeros_hbm, ones_hbm,
         z8_hbm, part_out, deg_out,
         sidxa, sidxb, didx2, rows, ones, zbuf, z8, acc, dacc,
         gsem, ssem, osem) = refs
    else:
        (xr_hbm, sa2_hbm, sb2_hbm, dst2_hbm, zeros_hbm,
         part_out,
         sidxa, sidxb, didx2, rows, zbuf, acc,
         gsem, ssem) = refs
    c = lax.axis_index("c")
    s = lax.axis_index("s")
    wid = s * NC + c
    row0 = s * RPT
    crow0 = wid * NCHUNK

    pltpu.sync_copy(zeros_hbm, zbuf)
    if with_deg:
        pltpu.sync_copy(ones_hbm, ones)
        pltpu.sync_copy(z8_hbm, z8)
    pltpu.sync_copy(sa2_hbm.at[pl.ds(crow0, NCHUNK)], sidxa)
    pltpu.sync_copy(sb2_hbm.at[pl.ds(crow0, NCHUNK)], sidxb)
    pltpu.sync_copy(dst2_hbm.at[pl.ds(crow0, NCHUNK)], didx2)
    for j in range(RPT // ZR):
        r = row0 + j * ZR
        pltpu.sync_copy(zbuf, acc.at[pl.ds(r, ZR)])
        if with_deg:
            pltpu.sync_copy(z8, dacc.at[pl.ds(r, ZR)])
    plsc.subcore_barrier()

    def run_pass(sidx2, deg_pass):
        for b in range(NBUF):
            pltpu.async_copy(xr_hbm.at[sidx2.at[b]], rows.at[b], gsem.at[b])

        def outer(it, carry):
            base = it * NBUF
            for b in range(NBUF):
                g = base + b
                pltpu.make_async_copy(
                    xr_hbm.at[sidx2.at[g]], rows.at[b], gsem.at[b]).wait()
                pltpu.async_copy(rows.at[b], acc.at[didx2.at[g]],
                                 ssem.at[b], add=True)
                if deg_pass:
                    pltpu.async_copy(ones, dacc.at[didx2.at[g]],
                                     osem.at[b], add=True)
            for b in range(NBUF):
                g = base + b
                pltpu.make_async_copy(rows.at[b], acc.at[didx2.at[g]],
                                      ssem.at[b]).wait()
                if deg_pass:
                    pltpu.make_async_copy(ones, dacc.at[didx2.at[g]],
                                          osem.at[b]).wait()

                @pl.when(g + NBUF < NCHUNK)
                def _():
                    pltpu.async_copy(xr_hbm.at[sidx2.at[g + NBUF]],
                                     rows.at[b], gsem.at[b])
            return carry

        lax.fori_loop(0, NCHUNK // NBUF, outer, 0)

    run_pass(sidxa, with_deg)
    plsc.subcore_barrier()

    for j in range(RPT // ZR):
        r = row0 + j * ZR
        pltpu.sync_copy(acc.at[pl.ds(r, ZR)], zbuf)
        pltpu.sync_copy(zbuf, part_out.at[c, pl.ds(r, ZR), pl.ds(0, DH)])
        if with_deg:
            pltpu.sync_copy(dacc.at[pl.ds(r, ZR)], z8)
            pltpu.sync_copy(z8, deg_out.at[c, pl.ds(r, ZR)])
    pltpu.sync_copy(zeros_hbm, zbuf)
    for j in range(RPT // ZR):
        r = row0 + j * ZR
        pltpu.sync_copy(zbuf, acc.at[pl.ds(r, ZR)])
    plsc.subcore_barrier()

    run_pass(sidxb, False)
    plsc.subcore_barrier()

    for j in range(RPT // ZR):
        r = row0 + j * ZR
        pltpu.sync_copy(acc.at[pl.ds(r, ZR)], zbuf)
        pltpu.sync_copy(zbuf, part_out.at[c, pl.ds(r, ZR), pl.ds(DH, DH)])


def _sc_agg_deg(xr, sa2, sb2, dst2, zeros_big, ones8, zeros8):
    out_type = (
        jax.ShapeDtypeStruct((NC, NP, D), jnp.float32),
        jax.ShapeDtypeStruct((NC, NP, DEGW), jnp.float32),
    )
    scratch = [
        pltpu.VMEM((NCHUNK, K), jnp.int32),
        pltpu.VMEM((NCHUNK, K), jnp.int32),
        pltpu.VMEM((NCHUNK, K), jnp.int32),
        pltpu.VMEM((NBUF, K, DH), jnp.float32),
        pltpu.VMEM((K, DEGW), jnp.float32),
        pltpu.VMEM((ZR, DH), jnp.float32),
        pltpu.VMEM((ZR, DEGW), jnp.float32),
        pltpu.VMEM_SHARED((NP, DH), jnp.float32),
        pltpu.VMEM_SHARED((NP, DEGW), jnp.float32),
        pltpu.SemaphoreType.DMA((NBUF,)),
        pltpu.SemaphoreType.DMA((NBUF,)),
        pltpu.SemaphoreType.DMA((NBUF,)),
    ]
    params = pltpu.CompilerParams(use_tc_tiling_on_sc=False)
    fn = functools.partial(_sc_agg_body, True)
    return pl.kernel(fn, mesh=_mesh, out_type=out_type,
                     scratch_types=scratch, compiler_params=params)(
        xr, sa2, sb2, dst2, zeros_big, ones8, zeros8)


def _sc_agg(xr, sa2, sb2, dst2, zeros_big):
    out_type = jax.ShapeDtypeStruct((NC, NP, D), jnp.float32)
    scratch = [
        pltpu.VMEM((NCHUNK, K), jnp.int32),
        pltpu.VMEM((NCHUNK, K), jnp.int32),
        pltpu.VMEM((NCHUNK, K), jnp.int32),
        pltpu.VMEM((NBUF, K, DH), jnp.float32),
        pltpu.VMEM((ZR, DH), jnp.float32),
        pltpu.VMEM_SHARED((NP, DH), jnp.float32),
        pltpu.SemaphoreType.DMA((NBUF,)),
        pltpu.SemaphoreType.DMA((NBUF,)),
    ]
    params = pltpu.CompilerParams(use_tc_tiling_on_sc=False)
    fn = functools.partial(_sc_agg_body, False)
    return pl.kernel(fn, mesh=_mesh, out_type=out_type,
                     scratch_types=scratch, compiler_params=params)(
        xr, sa2, sb2, dst2, zeros_big)


BN = 1000


def _tc0_body(part_ref, degp_ref, w_ref, b_ref, out_ref):
    p = part_ref[0] + part_ref[1]
    dg = degp_ref[0] + degp_ref[1]
    recip = 1.0 / jnp.maximum(dg[:, 0:1], 1.0)
    agg = p * recip
    h = jnp.dot(agg, w_ref[...], preferred_element_type=jnp.float32)
    out_ref[...] = jnp.maximum(h + b_ref[...], 0.0)


def _tc_layer0(part, degp, w, b2):
    grid = (N // BN,)
    return pl.pallas_call(
        _tc0_body,
        grid=grid,
        in_specs=[
            pl.BlockSpec((NC, BN, D), lambda i: (0, i, 0)),
            pl.BlockSpec((NC, BN, DEGW), lambda i: (0, i, 0)),
            pl.BlockSpec((D, D), lambda i: (0, 0)),
            pl.BlockSpec((1, D), lambda i: (0, 0)),
        ],
        out_specs=pl.BlockSpec((BN, D), lambda i: (i, 0)),
        out_shape=jax.ShapeDtypeStruct((N, D), jnp.float32),
    )(part, degp, w, b2)


def _tc1_body(part_ref, degp_ref, hx_ref, cx_ref, wg_ref, bg_ref,
              wx_ref, wh_ref, bsum_ref, hy_ref, cy_ref):
    p = part_ref[0] + part_ref[1]
    dg = degp_ref[0] + degp_ref[1]
    recip = 1.0 / jnp.maximum(dg[:, 0:1], 1.0)
    agg = p * recip
    xg = (jnp.dot(agg, wg_ref[...], preferred_element_type=jnp.float32)
          + bg_ref[...])
    gates = (jnp.dot(xg, wx_ref[...], preferred_element_type=jnp.float32)
             + jnp.dot(hx_ref[...], wh_ref[...], preferred_element_type=jnp.float32)
             + bsum_ref[...])
    ingate = jax.nn.sigmoid(gates[:, 0:D])
    forgetgate = jax.nn.sigmoid(gates[:, D:2 * D])
    cellgate = jnp.tanh(gates[:, 2 * D:3 * D])
    outgate = jax.nn.sigmoid(gates[:, 3 * D:4 * D])
    cy = cx_ref[...] * forgetgate + ingate * cellgate
    hy_ref[...] = outgate * jnp.tanh(cy)
    cy_ref[...] = cy


def _tc_final(part, degp, hx, cx, wg, bg2, wx, wh, bsum2):
    grid = (N // BN,)
    out_shape = (
        jax.ShapeDtypeStruct((N, D), jnp.float32),
        jax.ShapeDtypeStruct((N, D), jnp.float32),
    )
    return pl.pallas_call(
        _tc1_body,
        grid=grid,
        in_specs=[
            pl.BlockSpec((NC, BN, D), lambda i: (0, i, 0)),
            pl.BlockSpec((NC, BN, DEGW), lambda i: (0, i, 0)),
            pl.BlockSpec((BN, D), lambda i: (i, 0)),
            pl.BlockSpec((BN, D), lambda i: (i, 0)),
            pl.BlockSpec((D, D), lambda i: (0, 0)),
            pl.BlockSpec((1, D), lambda i: (0, 0)),
            pl.BlockSpec((D, 4 * D), lambda i: (0, 0)),
            pl.BlockSpec((D, 4 * D), lambda i: (0, 0)),
            pl.BlockSpec((1, 4 * D), lambda i: (0, 0)),
        ],
        out_specs=(
            pl.BlockSpec((BN, D), lambda i: (i, 0)),
            pl.BlockSpec((BN, D), lambda i: (i, 0)),
        ),
        out_shape=out_shape,
    )(part, degp, hx, cx, wg, bg2, wx, wh, bsum2)


def kernel(x, edge_index, hx, cx, W_g0, b_g0, W_g1, b_g1,
           W_x2h, b_x2h, W_h2h, b_h2h):
    se = edge_index[0] * 2
    sa2 = se.reshape(E // K, K)
    sb2 = (se + 1).reshape(E // K, K)
    dst2 = edge_index[1].reshape(E // K, K)
    xr = x.reshape(2 * N, DH)
    zeros_big = jnp.zeros((ZR, DH), jnp.float32)
    ones8 = jnp.ones((K, DEGW), jnp.float32)
    zeros8 = jnp.zeros((ZR, DEGW), jnp.float32)

    part0, degp = _sc_agg_deg(xr, sa2, sb2, dst2, zeros_big, ones8, zeros8)
    h1 = _tc_layer0(part0, degp, W_g0, b_g0.reshape(1, D))
    part1 = _sc_agg(h1.reshape(2 * N, DH), sa2, sb2, dst2, zeros_big)
    hy, cy = _tc_final(part1, degp, hx, cx, W_g1, b_g1.reshape(1, D),
                       W_x2h, W_h2h, (b_x2h + b_h2h).reshape(1, 4 * D))
    return (hy, cy)


# speedup vs baseline: 12.7299x; 1.0806x over previous
import functools

import jax
import jax.numpy as jnp
from jax import lax
from jax.experimental import pallas as pl
from jax.experimental.pallas import tpu as pltpu
from jax.experimental.pallas import tpu_sc as plsc

N = 10000
E = 320000
D = 128
DH = D // 2
NC = 2
NS = 16
NW = NC * NS
EPW = E // NW
K = 80
NCHUNK = EPW // K
NBUF = 5
NP = 10112
RPT = NP // NS
ZR = 79
DEGW = 8

_mesh = plsc.VectorSubcoreMesh(core_axis_name="c", subcore_axis_name="s")


def _sc_agg_body(with_deg, *refs):
    if with_deg:
        (xr_hbm, edges_hbm, zeros_hbm, ones_hbm,
         z8_hbm, part_out, deg_out,
         raw, sidxa, sidxb, didx2, rows, ones, zbuf, z8, acc, dacc,
         gsem, ssem, osem) = refs
    else:
        (xr_hbm, edges_hbm, zeros_hbm,
         part_out,
         raw, sidxa, sidxb, didx2, rows, zbuf, acc,
         gsem, ssem) = refs
    c = lax.axis_index("c")
    s = lax.axis_index("s")
    wid = s * NC + c
    row0 = s * RPT

    pltpu.sync_copy(zeros_hbm, zbuf)
    if with_deg:
        pltpu.sync_copy(ones_hbm, ones)
        pltpu.sync_copy(z8_hbm, z8)
    pltpu.sync_copy(edges_hbm.at[pl.ds(wid * EPW, EPW)], raw.at[0])
    pltpu.sync_copy(edges_hbm.at[pl.ds(E + wid * EPW, EPW)], raw.at[1])
    for j in range(RPT // ZR):
        r = row0 + j * ZR
        pltpu.sync_copy(zbuf, acc.at[pl.ds(r, ZR)])
        if with_deg:
            pltpu.sync_copy(z8, dacc.at[pl.ds(r, ZR)])

    def prep(r_i, carry):
        for j in range(K // 16):
            o = j * 16
            sv = raw[0, pl.ds(r_i * K + o, 16)]
            dv = raw[1, pl.ds(r_i * K + o, 16)]
            sidxa[r_i, pl.ds(o, 16)] = sv * 2
            sidxb[r_i, pl.ds(o, 16)] = sv * 2 + 1
            didx2[r_i, pl.ds(o, 16)] = dv
        return carry

    lax.fori_loop(0, NCHUNK, prep, 0)
    plsc.subcore_barrier()

    def run_pass(sidx2, deg_pass):
        for b in range(NBUF):
            pltpu.async_copy(xr_hbm.at[sidx2.at[b]], rows.at[b], gsem.at[b])

        def outer(it, carry):
            base = it * NBUF
            for b in range(NBUF):
                g = base + b
                pltpu.make_async_copy(
                    xr_hbm.at[sidx2.at[g]], rows.at[b], gsem.at[b]).wait()
                pltpu.async_copy(rows.at[b], acc.at[didx2.at[g]],
                                 ssem.at[b], add=True)
                if deg_pass:
                    pltpu.async_copy(ones, dacc.at[didx2.at[g]],
                                     osem.at[b], add=True)
            for b in range(NBUF):
                g = base + b
                pltpu.make_async_copy(rows.at[b], acc.at[didx2.at[g]],
                                      ssem.at[b]).wait()
                if deg_pass:
                    pltpu.make_async_copy(ones, dacc.at[didx2.at[g]],
                                          osem.at[b]).wait()

                @pl.when(g + NBUF < NCHUNK)
                def _():
                    pltpu.async_copy(xr_hbm.at[sidx2.at[g + NBUF]],
                                     rows.at[b], gsem.at[b])
            return carry

        lax.fori_loop(0, NCHUNK // NBUF, outer, 0)

    run_pass(sidxa, with_deg)
    plsc.subcore_barrier()

    for j in range(RPT // ZR):
        r = row0 + j * ZR
        pltpu.sync_copy(acc.at[pl.ds(r, ZR)], zbuf)
        pltpu.sync_copy(zbuf, part_out.at[c, pl.ds(r, ZR), pl.ds(0, DH)])
        if with_deg:
            pltpu.sync_copy(dacc.at[pl.ds(r, ZR)], z8)
            pltpu.sync_copy(z8, deg_out.at[c, pl.ds(r, ZR)])
    pltpu.sync_copy(zeros_hbm, zbuf)
    for j in range(RPT // ZR):
        r = row0 + j * ZR
        pltpu.sync_copy(zbuf, acc.at[pl.ds(r, ZR)])
    plsc.subcore_barrier()

    run_pass(sidxb, False)
    plsc.subcore_barrier()

    for j in range(RPT // ZR):
        r = row0 + j * ZR
        pltpu.sync_copy(acc.at[pl.ds(r, ZR)], zbuf)
        pltpu.sync_copy(zbuf, part_out.at[c, pl.ds(r, ZR), pl.ds(DH, DH)])


def _sc_agg_deg(xr, edges, zeros_big, ones8, zeros8):
    out_type = (
        jax.ShapeDtypeStruct((NC, NP, D), jnp.float32),
        jax.ShapeDtypeStruct((NC, NP, DEGW), jnp.float32),
    )
    scratch = [
        pltpu.VMEM((2, EPW), jnp.int32),
        pltpu.VMEM((NCHUNK, K), jnp.int32),
        pltpu.VMEM((NCHUNK, K), jnp.int32),
        pltpu.VMEM((NCHUNK, K), jnp.int32),
        pltpu.VMEM((NBUF, K, DH), jnp.float32),
        pltpu.VMEM((K, DEGW), jnp.float32),
        pltpu.VMEM((ZR, DH), jnp.float32),
        pltpu.VMEM((ZR, DEGW), jnp.float32),
        pltpu.VMEM_SHARED((NP, DH), jnp.float32),
        pltpu.VMEM_SHARED((NP, DEGW), jnp.float32),
        pltpu.SemaphoreType.DMA((NBUF,)),
        pltpu.SemaphoreType.DMA((NBUF,)),
        pltpu.SemaphoreType.DMA((NBUF,)),
    ]
    params = pltpu.CompilerParams(use_tc_tiling_on_sc=False)
    fn = functools.partial(_sc_agg_body, True)
    return pl.kernel(fn, mesh=_mesh, out_type=out_type,
                     scratch_types=scratch, compiler_params=params)(
        xr, edges, zeros_big, ones8, zeros8)


def _sc_agg(xr, edges, zeros_big):
    out_type = jax.ShapeDtypeStruct((NC, NP, D), jnp.float32)
    scratch = [
        pltpu.VMEM((2, EPW), jnp.int32),
        pltpu.VMEM((NCHUNK, K), jnp.int32),
        pltpu.VMEM((NCHUNK, K), jnp.int32),
        pltpu.VMEM((NCHUNK, K), jnp.int32),
        pltpu.VMEM((NBUF, K, DH), jnp.float32),
        pltpu.VMEM((ZR, DH), jnp.float32),
        pltpu.VMEM_SHARED((NP, DH), jnp.float32),
        pltpu.SemaphoreType.DMA((NBUF,)),
        pltpu.SemaphoreType.DMA((NBUF,)),
    ]
    params = pltpu.CompilerParams(use_tc_tiling_on_sc=False)
    fn = functools.partial(_sc_agg_body, False)
    return pl.kernel(fn, mesh=_mesh, out_type=out_type,
                     scratch_types=scratch, compiler_params=params)(
        xr, edges, zeros_big)


BN = 1000


def _tc0_body(part_ref, degp_ref, w_ref, b_ref, out_ref):
    p = part_ref[0] + part_ref[1]
    dg = degp_ref[0] + degp_ref[1]
    recip = 1.0 / jnp.maximum(dg[:, 0:1], 1.0)
    agg = p * recip
    h = jnp.dot(agg, w_ref[...], preferred_element_type=jnp.float32)
    out_ref[...] = jnp.maximum(h + b_ref[...], 0.0)


def _tc_layer0(part, degp, w, b2):
    grid = (N // BN,)
    return pl.pallas_call(
        _tc0_body,
        grid=grid,
        in_specs=[
            pl.BlockSpec((NC, BN, D), lambda i: (0, i, 0)),
            pl.BlockSpec((NC, BN, DEGW), lambda i: (0, i, 0)),
            pl.BlockSpec((D, D), lambda i: (0, 0)),
            pl.BlockSpec((1, D), lambda i: (0, 0)),
        ],
        out_specs=pl.BlockSpec((BN, D), lambda i: (i, 0)),
        out_shape=jax.ShapeDtypeStruct((N, D), jnp.float32),
    )(part, degp, w, b2)


def _tc1_body(part_ref, degp_ref, hx_ref, cx_ref, wg_ref, bg_ref,
              wx_ref, wh_ref, bsum_ref, hy_ref, cy_ref):
    p = part_ref[0] + part_ref[1]
    dg = degp_ref[0] + degp_ref[1]
    recip = 1.0 / jnp.maximum(dg[:, 0:1], 1.0)
    agg = p * recip
    xg = (jnp.dot(agg, wg_ref[...], preferred_element_type=jnp.float32)
          + bg_ref[...])
    bf = jnp.bfloat16
    gates = (jnp.dot(xg.astype(bf), wx_ref[...].astype(bf),
                     preferred_element_type=jnp.float32)
             + jnp.dot(hx_ref[...].astype(bf), wh_ref[...].astype(bf),
                       preferred_element_type=jnp.float32)
             + bsum_ref[...])
    ingate = jax.nn.sigmoid(gates[:, 0:D])
    forgetgate = jax.nn.sigmoid(gates[:, D:2 * D])
    cellgate = jnp.tanh(gates[:, 2 * D:3 * D])
    outgate = jax.nn.sigmoid(gates[:, 3 * D:4 * D])
    cy = cx_ref[...] * forgetgate + ingate * cellgate
    hy_ref[...] = outgate * jnp.tanh(cy)
    cy_ref[...] = cy


def _tc_final(part, degp, hx, cx, wg, bg2, wx, wh, bsum2):
    grid = (N // BN,)
    out_shape = (
        jax.ShapeDtypeStruct((N, D), jnp.float32),
        jax.ShapeDtypeStruct((N, D), jnp.float32),
    )
    return pl.pallas_call(
        _tc1_body,
        grid=grid,
        in_specs=[
            pl.BlockSpec((NC, BN, D), lambda i: (0, i, 0)),
            pl.BlockSpec((NC, BN, DEGW), lambda i: (0, i, 0)),
            pl.BlockSpec((BN, D), lambda i: (i, 0)),
            pl.BlockSpec((BN, D), lambda i: (i, 0)),
            pl.BlockSpec((D, D), lambda i: (0, 0)),
            pl.BlockSpec((1, D), lambda i: (0, 0)),
            pl.BlockSpec((D, 4 * D), lambda i: (0, 0)),
            pl.BlockSpec((D, 4 * D), lambda i: (0, 0)),
            pl.BlockSpec((1, 4 * D), lambda i: (0, 0)),
        ],
        out_specs=(
            pl.BlockSpec((BN, D), lambda i: (i, 0)),
            pl.BlockSpec((BN, D), lambda i: (i, 0)),
        ),
        out_shape=out_shape,
    )(part, degp, hx, cx, wg, bg2, wx, wh, bsum2)


def kernel(x, edge_index, hx, cx, W_g0, b_g0, W_g1, b_g1,
           W_x2h, b_x2h, W_h2h, b_h2h):
    edges = edge_index.reshape(2 * E)
    xr = x.reshape(2 * N, DH)
    zeros_big = jnp.zeros((ZR, DH), jnp.float32)
    ones8 = jnp.ones((K, DEGW), jnp.float32)
    zeros8 = jnp.zeros((ZR, DEGW), jnp.float32)

    part0, degp = _sc_agg_deg(xr, edges, zeros_big, ones8, zeros8)
    h1 = _tc_layer0(part0, degp, W_g0, b_g0.reshape(1, D))
    part1 = _sc_agg(h1.reshape(2 * N, DH), edges, zeros_big)
    hy, cy = _tc_final(part1, degp, hx, cx, W_g1, b_g1.reshape(1, D),
                       W_x2h, W_h2h, (b_x2h + b_h2h).reshape(1, 4 * D))
    return (hy, cy)


# speedup vs baseline: 12.8523x; 1.0096x over previous
import functools

import jax
import jax.numpy as jnp
from jax import lax
from jax.experimental import pallas as pl
from jax.experimental.pallas import tpu as pltpu
from jax.experimental.pallas import tpu_sc as plsc

N = 10000
E = 320000
D = 128
DH = D // 2
NC = 2
NS = 16
NW = NC * NS
EPW = E // NW
K = 80
NCHUNK = EPW // K
NBUF = 5
NP = 10112
RPT = NP // NS
ZR = 79
DEGW = 8

_mesh = plsc.VectorSubcoreMesh(core_axis_name="c", subcore_axis_name="s")


def _sc_agg_body(with_deg, *refs):
    if with_deg:
        (xr_hbm, edges_hbm, zeros_hbm, ones_hbm,
         z8_hbm, deg_out, part_out,
         raw, sidxa, sidxb, didx2, rows, ones, zbuf, z8, acc, dacc,
         gsem, ssem, osem) = refs
    else:
        (xr_hbm, edges_hbm, zeros_hbm,
         part_out,
         raw, sidxa, sidxb, didx2, rows, zbuf, acc,
         gsem, ssem) = refs
    c = lax.axis_index("c")
    s = lax.axis_index("s")
    wid = s * NC + c
    row0 = s * RPT

    pltpu.sync_copy(zeros_hbm, zbuf)
    if with_deg:
        pltpu.sync_copy(ones_hbm, ones)
        pltpu.sync_copy(z8_hbm, z8)
    pltpu.sync_copy(edges_hbm.at[pl.ds(wid * EPW, EPW)], raw.at[0])
    pltpu.sync_copy(edges_hbm.at[pl.ds(E + wid * EPW, EPW)], raw.at[1])
    for j in range(RPT // ZR):
        r = row0 + j * ZR
        pltpu.sync_copy(zbuf, acc.at[pl.ds(r, ZR)])
        if with_deg:
            pltpu.sync_copy(z8, dacc.at[pl.ds(r, ZR)])

    def prep(r_i, carry):
        for j in range(K // 16):
            o = j * 16
            sv = raw[0, pl.ds(r_i * K + o, 16)]
            dv = raw[1, pl.ds(r_i * K + o, 16)]
            sidxa[r_i, pl.ds(o, 16)] = sv * 2
            sidxb[r_i, pl.ds(o, 16)] = sv * 2 + 1
            didx2[r_i, pl.ds(o, 16)] = dv
        return carry

    lax.fori_loop(0, NCHUNK, prep, 0)
    plsc.subcore_barrier()

    def run_pass(sidx2, deg_pass):
        for b in range(NBUF):
            pltpu.async_copy(xr_hbm.at[sidx2.at[b]], rows.at[b], gsem.at[b])

        def outer(it, carry):
            base = it * NBUF
            for b in range(NBUF):
                g = base + b
                pltpu.make_async_copy(
                    xr_hbm.at[sidx2.at[g]], rows.at[b], gsem.at[b]).wait()
                pltpu.async_copy(rows.at[b], acc.at[didx2.at[g]],
                                 ssem.at[b], add=True)
                if deg_pass:
                    pltpu.async_copy(ones, dacc.at[didx2.at[g]],
                                     osem.at[b], add=True)
            for b in range(NBUF):
                g = base + b
                pltpu.make_async_copy(rows.at[b], acc.at[didx2.at[g]],
                                      ssem.at[b]).wait()
                if deg_pass:
                    pltpu.make_async_copy(ones, dacc.at[didx2.at[g]],
                                          osem.at[b]).wait()

                @pl.when(g + NBUF < NCHUNK)
                def _():
                    pltpu.async_copy(xr_hbm.at[sidx2.at[g + NBUF]],
                                     rows.at[b], gsem.at[b])
            return carry

        lax.fori_loop(0, NCHUNK // NBUF, outer, 0)

    run_pass(sidxa, with_deg)
    plsc.subcore_barrier()

    for j in range(RPT // ZR):
        r = row0 + j * ZR
        pltpu.sync_copy(acc.at[pl.ds(r, ZR)], zbuf)
        pltpu.sync_copy(zbuf, part_out.at[c, pl.ds(r, ZR), pl.ds(0, DH)])
        if with_deg:
            pltpu.sync_copy(dacc.at[pl.ds(r, ZR)], z8)
            pltpu.sync_copy(z8, deg_out.at[c, pl.ds(r, ZR)])
    pltpu.sync_copy(zeros_hbm, zbuf)
    for j in range(RPT // ZR):
        r = row0 + j * ZR
        pltpu.sync_copy(zbuf, acc.at[pl.ds(r, ZR)])
    plsc.subcore_barrier()

    run_pass(sidxb, False)
    plsc.subcore_barrier()

    for j in range(RPT // ZR):
        r = row0 + j * ZR
        pltpu.sync_copy(acc.at[pl.ds(r, ZR)], zbuf)
        pltpu.sync_copy(zbuf, part_out.at[c, pl.ds(r, ZR), pl.ds(DH, DH)])


def _sc_agg_deg(xr, edges, zeros_big, ones8, zeros8):
    out_type = (
        jax.ShapeDtypeStruct((NC, NP, DEGW), jnp.float32),
        jax.ShapeDtypeStruct((NC, NP, D), jnp.float32),
    )
    scratch = [
        pltpu.VMEM((2, EPW), jnp.int32),
        pltpu.VMEM((NCHUNK, K), jnp.int32),
        pltpu.VMEM((NCHUNK, K), jnp.int32),
        pltpu.VMEM((NCHUNK, K), jnp.int32),
        pltpu.VMEM((NBUF, K, DH), jnp.float32),
        pltpu.VMEM((K, DEGW), jnp.float32),
        pltpu.VMEM((ZR, DH), jnp.float32),
        pltpu.VMEM((ZR, DEGW), jnp.float32),
        pltpu.VMEM_SHARED((NP, DH), jnp.float32),
        pltpu.VMEM_SHARED((NP, DEGW), jnp.float32),
        pltpu.SemaphoreType.DMA((NBUF,)),
        pltpu.SemaphoreType.DMA((NBUF,)),
        pltpu.SemaphoreType.DMA((NBUF,)),
    ]
    params = pltpu.CompilerParams(use_tc_tiling_on_sc=False)
    fn = functools.partial(_sc_agg_body, True)
    return pl.kernel(fn, mesh=_mesh, out_type=out_type,
                     scratch_types=scratch, compiler_params=params)(
        xr, edges, zeros_big, ones8, zeros8)


def _sc_agg(xr, edges, zeros_big):
    out_type = jax.ShapeDtypeStruct((NC, NP, D), jnp.float32)
    scratch = [
        pltpu.VMEM((2, EPW), jnp.int32),
        pltpu.VMEM((NCHUNK, K), jnp.int32),
        pltpu.VMEM((NCHUNK, K), jnp.int32),
        pltpu.VMEM((NCHUNK, K), jnp.int32),
        pltpu.VMEM((NBUF, K, DH), jnp.float32),
        pltpu.VMEM((ZR, DH), jnp.float32),
        pltpu.VMEM_SHARED((NP, DH), jnp.float32),
        pltpu.SemaphoreType.DMA((NBUF,)),
        pltpu.SemaphoreType.DMA((NBUF,)),
    ]
    params = pltpu.CompilerParams(use_tc_tiling_on_sc=False)
    fn = functools.partial(_sc_agg_body, False)
    return pl.kernel(fn, mesh=_mesh, out_type=out_type,
                     scratch_types=scratch, compiler_params=params)(
        xr, edges, zeros_big)


BN = 2000


def _tc0_body(part_ref, degp_ref, w_ref, b_ref, out_ref):
    p = part_ref[0] + part_ref[1]
    dg = degp_ref[0] + degp_ref[1]
    recip = 1.0 / jnp.maximum(dg[:, 0:1], 1.0)
    agg = p * recip
    h = jnp.dot(agg, w_ref[...], preferred_element_type=jnp.float32)
    out_ref[...] = jnp.maximum(h + b_ref[...], 0.0)


def _tc_layer0(part, degp, w, b2):
    grid = (N // BN,)
    return pl.pallas_call(
        _tc0_body,
        grid=grid,
        in_specs=[
            pl.BlockSpec((NC, BN, D), lambda i: (0, i, 0)),
            pl.BlockSpec((NC, BN, DEGW), lambda i: (0, i, 0)),
            pl.BlockSpec((D, D), lambda i: (0, 0)),
            pl.BlockSpec((1, D), lambda i: (0, 0)),
        ],
        out_specs=pl.BlockSpec((BN, D), lambda i: (i, 0)),
        out_shape=jax.ShapeDtypeStruct((N, D), jnp.float32),
    )(part, degp, w, b2)


def _tc1_body(part_ref, degp_ref, hx_ref, cx_ref, wg_ref, bg_ref,
              wx_ref, wh_ref, bsum_ref, hy_ref, cy_ref):
    p = part_ref[0] + part_ref[1]
    dg = degp_ref[0] + degp_ref[1]
    recip = 1.0 / jnp.maximum(dg[:, 0:1], 1.0)
    agg = p * recip
    xg = (jnp.dot(agg, wg_ref[...], preferred_element_type=jnp.float32)
          + bg_ref[...])
    bf = jnp.bfloat16
    gates = (jnp.dot(xg.astype(bf), wx_ref[...].astype(bf),
                     preferred_element_type=jnp.float32)
             + jnp.dot(hx_ref[...].astype(bf), wh_ref[...].astype(bf),
                       preferred_element_type=jnp.float32)
             + bsum_ref[...])
    ingate = jax.nn.sigmoid(gates[:, 0:D])
    forgetgate = jax.nn.sigmoid(gates[:, D:2 * D])
    cellgate = jnp.tanh(gates[:, 2 * D:3 * D])
    outgate = jax.nn.sigmoid(gates[:, 3 * D:4 * D])
    cy = cx_ref[...] * forgetgate + ingate * cellgate
    hy_ref[...] = outgate * jnp.tanh(cy)
    cy_ref[...] = cy


def _tc_final(part, degp, hx, cx, wg, bg2, wx, wh, bsum2):
    grid = (N // BN,)
    out_shape = (
        jax.ShapeDtypeStruct((N, D), jnp.float32),
        jax.ShapeDtypeStruct((N, D), jnp.float32),
    )
    return pl.pallas_call(
        _tc1_body,
        grid=grid,
        in_specs=[
            pl.BlockSpec((NC, BN, D), lambda i: (0, i, 0)),
            pl.BlockSpec((NC, BN, DEGW), lambda i: (0, i, 0)),
            pl.BlockSpec((BN, D), lambda i: (i, 0)),
            pl.BlockSpec((BN, D), lambda i: (i, 0)),
            pl.BlockSpec((D, D), lambda i: (0, 0)),
            pl.BlockSpec((1, D), lambda i: (0, 0)),
            pl.BlockSpec((D, 4 * D), lambda i: (0, 0)),
            pl.BlockSpec((D, 4 * D), lambda i: (0, 0)),
            pl.BlockSpec((1, 4 * D), lambda i: (0, 0)),
        ],
        out_specs=(
            pl.BlockSpec((BN, D), lambda i: (i, 0)),
            pl.BlockSpec((BN, D), lambda i: (i, 0)),
        ),
        out_shape=out_shape,
    )(part, degp, hx, cx, wg, bg2, wx, wh, bsum2)


def kernel(x, edge_index, hx, cx, W_g0, b_g0, W_g1, b_g1,
           W_x2h, b_x2h, W_h2h, b_h2h):
    edges = edge_index.reshape(2 * E)
    xr = x.reshape(2 * N, DH)
    zeros_big = jnp.zeros((ZR, DH), jnp.float32)
    ones8 = jnp.ones((K, DEGW), jnp.float32)
    zeros8 = jnp.zeros((ZR, DEGW), jnp.float32)

    degp, part0 = _sc_agg_deg(xr, edges, zeros_big, ones8, zeros8)
    h1 = _tc_layer0(part0, degp, W_g0, b_g0.reshape(1, D))
    part1 = _sc_agg(h1.reshape(2 * N, DH), edges, zeros_big)
    hy, cy = _tc_final(part1, degp, hx, cx, W_g1, b_g1.reshape(1, D),
                       W_x2h, W_h2h, (b_x2h + b_h2h).reshape(1, 4 * D))
    return (hy, cy)
